# baseline (speedup 1.0000x reference)
import functools
import math

import jax
import jax.numpy as jnp
import numpy as np
from jax import lax
from jax.experimental import pallas as pl
from jax.experimental.pallas import tpu as pltpu
from jax.experimental.pallas import tpu_sc as plsc

VOCAB = 100000
SEQ_LEN = 2048
D_MODEL = 768
BATCH = 4
SCALE = math.sqrt(float(D_MODEL))

NUM_WORKERS = 32
SEQ_PER_W = SEQ_LEN // NUM_WORKERS
CHUNK = 32
NCHUNKS = BATCH * SEQ_PER_W // CHUNK
LANES = 16
CHUNKS_PER_ROW = D_MODEL // LANES


def _make_pe() -> np.ndarray:
    pos = np.arange(SEQ_LEN, dtype=np.float32)[:, None]
    div = np.exp(
        np.arange(0, D_MODEL, 2, dtype=np.float32)
        * (-math.log(10000.0) / D_MODEL)
    )
    pe = np.zeros((SEQ_LEN, D_MODEL), dtype=np.float32)
    pe[:, 0::2] = np.sin(pos * div)
    pe[:, 1::2] = np.cos(pos * div)
    return pe


_PE = jnp.asarray(_make_pe())

_mesh = plsc.VectorSubcoreMesh(core_axis_name="c", subcore_axis_name="s")


@functools.partial(
    pl.kernel,
    mesh=_mesh,
    out_type=jax.ShapeDtypeStruct((BATCH * SEQ_LEN, D_MODEL), jnp.float32),
    scratch_types=[
        pltpu.VMEM((BATCH, SEQ_PER_W), jnp.int32),
        pltpu.VMEM((SEQ_PER_W, D_MODEL), jnp.float32),
        pltpu.VMEM((CHUNK, D_MODEL), jnp.float32),
        pltpu.VMEM((CHUNK, D_MODEL), jnp.float32),
        pltpu.SemaphoreType.DMA,
        pltpu.SemaphoreType.DMA,
        pltpu.SemaphoreType.DMA,
        pltpu.SemaphoreType.DMA,
        pltpu.SemaphoreType.DMA,
    ],
)
def _encode(tokens_hbm, pe_hbm, table_hbm, out_hbm,
            idx_v, pe_v, buf0, buf1, pe_sem, g0, g1, w0, w1):
    wid = lax.axis_index("s") * 2 + lax.axis_index("c")
    seq_base = wid * SEQ_PER_W
    bufs = (buf0, buf1)
    gsems = (g0, g1)
    wsems = (w0, w1)

    pe_cp = pltpu.async_copy(pe_hbm.at[pl.ds(seq_base, SEQ_PER_W)], pe_v,
                             pe_sem)
    for b in range(BATCH):
        pltpu.sync_copy(
            tokens_hbm.at[pl.ds(b * SEQ_LEN + seq_base, SEQ_PER_W)],
            idx_v.at[b])

    def idx_ref(k):
        b, h = divmod(k, SEQ_PER_W // CHUNK)
        return idx_v.at[b, pl.ds(h * CHUNK, CHUNK)]

    def out_slice(k):
        b, h = divmod(k, SEQ_PER_W // CHUNK)
        return out_hbm.at[pl.ds(b * SEQ_LEN + seq_base + h * CHUNK, CHUNK)]

    gathers = [None] * NCHUNKS
    writes = [None] * NCHUNKS
    gathers[0] = pltpu.async_copy(table_hbm.at[idx_ref(0)], bufs[0],
                                  gsems[0])
    pe_cp.wait()
    for k in range(NCHUNKS):
        cur = k % 2
        gathers[k].wait()
        if k + 1 < NCHUNKS:
            if k >= 1:
                writes[k - 1].wait()
            gathers[k + 1] = pltpu.async_copy(
                table_hbm.at[idx_ref(k + 1)], bufs[1 - cur],
                gsems[1 - cur])
        buf = bufs[cur]
        pe_base = (k % (SEQ_PER_W // CHUNK)) * CHUNK

        def body(r, carry):
            for c in range(CHUNKS_PER_ROW):
                sl = pl.ds(c * LANES, LANES)
                buf[r, sl] = buf[r, sl] * SCALE + pe_v[pe_base + r, sl]
            return carry

        lax.fori_loop(0, CHUNK, body, 0)
        writes[k] = pltpu.async_copy(buf, out_slice(k), wsems[cur])
    writes[NCHUNKS - 2].wait()
    writes[NCHUNKS - 1].wait()


def kernel(tokens, table):
    tokens_flat = tokens.reshape(-1).astype(jnp.int32)
    out = _encode(tokens_flat, _PE, table)
    return out.reshape(BATCH, SEQ_LEN, D_MODEL)


# speedup vs baseline: 1.3677x; 1.3677x over previous
import functools
import math

import jax
import jax.numpy as jnp
import numpy as np
from jax import lax
from jax.experimental import pallas as pl
from jax.experimental.pallas import tpu as pltpu
from jax.experimental.pallas import tpu_sc as plsc

VOCAB = 100000
SEQ_LEN = 2048
D_MODEL = 768
BATCH = 4
SCALE = math.sqrt(float(D_MODEL))

NUM_WORKERS = 32
SEQ_PER_W = SEQ_LEN // NUM_WORKERS
CHUNK = 32
NCHUNKS = BATCH * SEQ_PER_W // CHUNK
NBUF = 3
LANES = 16
CHUNKS_PER_ROW = D_MODEL // LANES


def _make_pe() -> np.ndarray:
    pos = np.arange(SEQ_LEN, dtype=np.float32)[:, None]
    div = np.exp(
        np.arange(0, D_MODEL, 2, dtype=np.float32)
        * (-math.log(10000.0) / D_MODEL)
    )
    pe = np.zeros((SEQ_LEN, D_MODEL), dtype=np.float32)
    pe[:, 0::2] = np.sin(pos * div)
    pe[:, 1::2] = np.cos(pos * div)
    return pe


_PE = jnp.asarray(_make_pe())

_mesh = plsc.VectorSubcoreMesh(core_axis_name="c", subcore_axis_name="s")


@functools.partial(
    pl.kernel,
    mesh=_mesh,
    out_type=jax.ShapeDtypeStruct((BATCH * SEQ_LEN, D_MODEL), jnp.float32),
    scratch_types=[
        pltpu.VMEM((BATCH, SEQ_PER_W), jnp.int32),
        pltpu.VMEM((SEQ_PER_W, D_MODEL), jnp.float32),
        pltpu.VMEM((CHUNK, D_MODEL), jnp.float32),
        pltpu.VMEM((CHUNK, D_MODEL), jnp.float32),
        pltpu.VMEM((CHUNK, D_MODEL), jnp.float32),
        pltpu.SemaphoreType.DMA,
        pltpu.SemaphoreType.DMA,
        pltpu.SemaphoreType.DMA,
        pltpu.SemaphoreType.DMA,
        pltpu.SemaphoreType.DMA,
        pltpu.SemaphoreType.DMA,
        pltpu.SemaphoreType.DMA,
    ],
)
def _encode(tokens_hbm, pe_hbm, table_hbm, out_hbm,
            idx_v, pe_v, buf0, buf1, buf2,
            pe_sem, g0, g1, g2, w0, w1, w2):
    wid = lax.axis_index("s") * 2 + lax.axis_index("c")
    seq_base = wid * SEQ_PER_W
    bufs = (buf0, buf1, buf2)
    gsems = (g0, g1, g2)
    wsems = (w0, w1, w2)

    pe_cp = pltpu.async_copy(pe_hbm.at[pl.ds(seq_base, SEQ_PER_W)], pe_v,
                             pe_sem)
    for b in range(BATCH):
        pltpu.sync_copy(
            tokens_hbm.at[pl.ds(b * SEQ_LEN + seq_base, SEQ_PER_W)],
            idx_v.at[b])

    def idx_ref(k):
        b, h = divmod(k, SEQ_PER_W // CHUNK)
        return idx_v.at[b, pl.ds(h * CHUNK, CHUNK)]

    def out_slice(k):
        b, h = divmod(k, SEQ_PER_W // CHUNK)
        return out_hbm.at[pl.ds(b * SEQ_LEN + seq_base + h * CHUNK, CHUNK)]

    gathers = [None] * NCHUNKS
    writes = [None] * NCHUNKS
    gathers[0] = pltpu.async_copy(table_hbm.at[idx_ref(0)], bufs[0],
                                  gsems[0])
    pe_cp.wait()
    for k in range(NCHUNKS):
        cur = k % NBUF
        gathers[k].wait()
        if k + 1 < NCHUNKS:
            nxt = (k + 1) % NBUF
            if k >= NBUF - 1:
                writes[k - (NBUF - 1)].wait()
            gathers[k + 1] = pltpu.async_copy(
                table_hbm.at[idx_ref(k + 1)], bufs[nxt], gsems[nxt])
        buf = bufs[cur]
        pe_base = (k % (SEQ_PER_W // CHUNK)) * CHUNK

        @plsc.parallel_loop(0, CHUNK, step=1, unroll=2)
        def _row(r):
            for c in range(CHUNKS_PER_ROW):
                sl = pl.ds(c * LANES, LANES)
                buf[r, sl] = buf[r, sl] * SCALE + pe_v[pe_base + r, sl]

        writes[k] = pltpu.async_copy(buf, out_slice(k), wsems[cur])
    for k in range(NCHUNKS - (NBUF - 1), NCHUNKS):
        writes[k].wait()


def kernel(tokens, table):
    tokens_flat = tokens.reshape(-1).astype(jnp.int32)
    out = _encode(tokens_flat, _PE, table)
    return out.reshape(BATCH, SEQ_LEN, D_MODEL)
